# baseline (speedup 1.0000x reference)
import functools

import jax
import jax.numpy as jnp
from jax import lax
from jax.experimental import pallas as pl
from jax.experimental.pallas import tpu as pltpu
from jax.experimental.pallas import tpu_sc as plsc

NODE_SIZES = [131072, 32768, 8192, 2048, 512, 128]
CHANNEL_SIZES = [1, 8, 16, 32, 64, 128]
NS = 16
HALF = 16


CH = 512
G = 128


def _sc_mesh():
    return plsc.VectorSubcoreMesh(core_axis_name="c", subcore_axis_name="s",
                                  num_cores=2, num_subcores=NS)


@functools.cache
def _sc_seg(n_in: int, n_out: int, k: int, m: int):
    w = k * m
    np_ = 1
    while (n_out // (NS * np_) + 1) * w > 100000:
        np_ *= 2
    rows_r = n_out // (NS * np_)
    nch = n_in // G

    def body(in_ref, dst_ref, out_ref, table, idxw, ldst, dbuf):
        c = lax.axis_index("c")
        s = lax.axis_index("s")
        zero16 = jnp.zeros((16,), jnp.float32)

        for p in range(np_):
            lo = (s * np_ + p) * rows_r

            def zf(i, _):
                zr = i // (w // 16)
                zk = i % (w // 16)
                table[zr, pl.ds(zk * 16, 16)] = zero16
                return _

            lax.fori_loop(0, (rows_r + 1) * (w // 16), zf, None)

            def chunk(chi, _a):
                pltpu.sync_copy(dst_ref.at[chi, :], idxw)
                for u in range(G // 16):
                    v = idxw[pl.ds(u * 16, 16)]
                    mask = (v >= lo) & (v < lo + rows_r)
                    ldst[u, :] = jnp.where(mask, v - lo, rows_r)
                for ki in range(k):
                    pltpu.sync_copy(
                        in_ref.at[c, ki, pl.ds(chi * G, G), :], dbuf)

                    def acc(u, _b):
                        ld_vec = ldst[u, :]
                        for t in range(16):
                            ld = ld_vec[t]
                            for kk in range(m // 16):
                                cs = ki * m + kk * 16
                                cur = table[ld, pl.ds(cs, 16)]
                                table[ld, pl.ds(cs, 16)] = (
                                    cur
                                    + dbuf[u * 16 + t, pl.ds(kk * 16, 16)])
                        return _b

                    lax.fori_loop(0, G // 16, acc, None)
                return _a

            lax.fori_loop(0, nch, chunk, None)

            for ki in range(k):
                pltpu.sync_copy(
                    table.at[pl.ds(0, rows_r), pl.ds(ki * m, m)],
                    out_ref.at[c, ki, pl.ds(lo, rows_r), :])

    return pl.kernel(
        body,
        out_type=jax.ShapeDtypeStruct((2, k, n_out, m), jnp.float32),
        mesh=_sc_mesh(),
        scratch_types=[
            pltpu.VMEM((rows_r + 1, w), jnp.float32),
            pltpu.VMEM((G,), jnp.int32),
            pltpu.VMEM((G // 16, 16), jnp.int32),
            pltpu.VMEM((G, m), jnp.float32),
        ],
    )


def _transpose_body(x_ref, w_ref, o_ref):
    o_ref[0, 0] = lax.dot_general(x_ref[...], w_ref[...],
                                  (((0,), (0,)), ((), ())),
                                  preferred_element_type=jnp.float32)


def _transpose_x(x, w0t):
    n = x.shape[1]
    bn = 8192
    return pl.pallas_call(
        _transpose_body,
        grid=(2, n // bn),
        in_specs=[pl.BlockSpec((HALF, bn), lambda h, j: (h, j)),
                  pl.BlockSpec((HALF, 8 * HALF), lambda h, j: (0, 0))],
        out_specs=pl.BlockSpec((1, 1, bn, 8 * HALF), lambda h, j: (h, 0, j, 0)),
        out_shape=jax.ShapeDtypeStruct((2, 1, n, 8 * HALF), jnp.float32),
    )(x, w0t)


def _mix_body(a_ref, w_ref, b_ref, o_ref, *, k_in, k_out):
    acc = None
    for ki in range(k_in):
        part = lax.dot_general(a_ref[0, ki], w_ref[ki],
                               (((1,), (0,)), ((), ())),
                               preferred_element_type=jnp.float32)
        acc = part if acc is None else acc + part
    y = acc + b_ref[0][None, :]
    y = jnp.where(y >= 0, y, 0.2 * y)
    for ko in range(k_out):
        o_ref[0, ko] = y[:, ko * 128:(ko + 1) * 128]


def _mix(agg, w_t, b_big, bn):
    _, k_in, n, m_in = agg.shape
    w_out = w_t.shape[2]
    k_out = w_out // 128
    body = functools.partial(_mix_body, k_in=k_in, k_out=k_out)
    return pl.pallas_call(
        body,
        grid=(2, n // bn),
        in_specs=[
            pl.BlockSpec((1, k_in, bn, m_in), lambda h, j: (h, 0, j, 0)),
            pl.BlockSpec((k_in, m_in, w_out), lambda h, j: (0, 0, 0)),
            pl.BlockSpec((1, w_out), lambda h, j: (0, 0)),
        ],
        out_specs=pl.BlockSpec((1, k_out, bn, 128), lambda h, j: (h, 0, j, 0)),
        out_shape=jax.ShapeDtypeStruct((2, k_out, n, 128), jnp.float32),
    )(agg, w_t, b_big)


def _final_body(a_ref, w_ref, b_ref, m_ref, g_ref, brf_ref, o_ref, *, k_in):
    rows = []
    for h in range(2):
        acc = None
        for ki in range(k_in):
            part = lax.dot_general(a_ref[h, ki], w_ref[ki],
                                   (((1,), (0,)), ((), ())),
                                   preferred_element_type=jnp.float32)
            acc = part if acc is None else acc + part
        y = acc + b_ref[0][None, :]
        y = jnp.where(y >= 0, y, 0.2 * y)
        p = y * m_ref[...]
        rows.append(jnp.sum(p, axis=0, keepdims=True))
    acc = jnp.concatenate(rows, axis=0)
    rf = lax.dot_general(acc, g_ref[...], (((1,), (0,)), ((), ())),
                         preferred_element_type=jnp.float32)
    o_ref[...] = 1.0 / (1.0 + jnp.exp(-(rf + brf_ref[0, 0])))


def _final(agg4, w_t, b_big, m_full, g_sel, b_rf):
    k_in = agg4.shape[1]
    body = functools.partial(_final_body, k_in=k_in)
    return pl.pallas_call(
        body,
        grid=(1,),
        in_specs=[
            pl.BlockSpec(agg4.shape, lambda i: (0, 0, 0, 0)),
            pl.BlockSpec(w_t.shape, lambda i: (0, 0, 0)),
            pl.BlockSpec((1, w_t.shape[2]), lambda i: (0, 0)),
            pl.BlockSpec(m_full.shape, lambda i: (0, 0)),
            pl.BlockSpec(g_sel.shape, lambda i: (0, 0)),
            pl.BlockSpec((1, 1), lambda i: (0, 0)),
        ],
        out_specs=pl.BlockSpec((2, HALF), lambda i: (0, 0)),
        out_shape=jax.ShapeDtypeStruct((2, HALF), jnp.float32),
    )(agg4, w_t, b_big, m_full, g_sel, b_rf)


def kernel(x, dst0, dst1, dst2, dst3, dst4,
           W0, b0, W1, b1, W2, b2, W3, b3, W4, b4, W_rf, b_rf):
    dsts = [dst0, dst1, dst2, dst3, dst4]
    Ws = [W0, W1, W2, W3, W4]
    bs = [b0, b1, b2, b3, b4]

    eye = jnp.eye(HALF, dtype=jnp.float32)
    w0t = jnp.transpose(jnp.kron(eye, W0))
    cur = _transpose_x(x, w0t)
    bns = [4096, 2048, 1024, 512, 128]
    for l in range(5):
        n_out = NODE_SIZES[l + 1]
        _, k, _, m = cur.shape
        agg = _sc_seg(NODE_SIZES[l], n_out, k, m)(
            cur, dsts[l].reshape(-1, 128))
        w_out = HALF * CHANNEL_SIZES[l + 1]
        if l == 0:
            w_t = jnp.eye(w_out, dtype=jnp.float32)[None, :, :]
        else:
            w_in = HALF * CHANNEL_SIZES[l]
            m_in = min(128, w_in)
            k_in = w_in // m_in
            w_big = jnp.kron(eye, Ws[l])
            w_t = jnp.transpose(w_big).reshape(k_in, m_in, w_out)
        b_big = jnp.tile(bs[l], HALF)[None, :]
        if l < 4:
            cur = _mix(agg, w_t, b_big, bns[l])
        else:
            n4, c4 = NODE_SIZES[5], CHANNEL_SIZES[5]
            wrf = W_rf[:, 0].reshape(c4, n4)
            m_full = jnp.tile(jnp.transpose(wrf), (1, HALF))
            g_sel = jnp.repeat(jnp.eye(HALF, dtype=jnp.float32),
                               c4, axis=0)
            rf = _final(agg, w_t, b_big, m_full, g_sel, b_rf.reshape(1, 1))
    rf = rf.reshape(32, 1)
    neg = jnp.full((32,), -1.0, dtype=jnp.float32)
    return (rf, neg, neg, neg)


# speedup vs baseline: 1.2421x; 1.2421x over previous
import functools

import jax
import jax.numpy as jnp
from jax import lax
from jax.experimental import pallas as pl
from jax.experimental.pallas import tpu as pltpu
from jax.experimental.pallas import tpu_sc as plsc

NODE_SIZES = [131072, 32768, 8192, 2048, 512, 128]
CHANNEL_SIZES = [1, 8, 16, 32, 64, 128]
NS = 16
HALF = 16


CH = 512
G = 128


def _sc_mesh():
    return plsc.VectorSubcoreMesh(core_axis_name="c", subcore_axis_name="s",
                                  num_cores=2, num_subcores=NS)


@functools.cache
def _sc_seg(n_in: int, n_out: int, k: int, m: int):
    w = k * m
    np_ = 1
    while (n_out // (NS * np_) + 1) * w > 100000:
        np_ *= 2
    rows_r = n_out // (NS * np_)
    nch = n_in // G

    def body(in_ref, dst_ref, out_ref, table, idxw, dbuf):
        c = lax.axis_index("c")
        s = lax.axis_index("s")
        zero16 = jnp.zeros((16,), jnp.float32)

        def one_pass(p, _z):
            lo = (s * np_ + p) * rows_r

            def zf(i, _):
                q = i // (m // 16)
                zk = i % (m // 16)
                kz = q // (rows_r + 1)
                zr = q % (rows_r + 1)
                table[kz, zr, pl.ds(zk * 16, 16)] = zero16
                return _

            lax.fori_loop(0, k * (rows_r + 1) * (m // 16), zf, None)

            def chunk(chi, _a):
                pltpu.sync_copy(dst_ref.at[chi, :], idxw)
                ldsc, hit_u = [], []
                for u in range(G // 16):
                    v = idxw[pl.ds(u * 16, 16)]
                    mu = (v >= lo) & (v < lo + rows_r)
                    ldv = jnp.where(mu, v - lo, rows_r)
                    row = [ldv[t] for t in range(16)]
                    ldsc.append(row)
                    hu = row[0] < rows_r
                    for t in range(1, 16):
                        hu = hu | (row[t] < rows_r)
                    hit_u.append(hu)
                any_hit = hit_u[0]
                for u in range(1, G // 16):
                    any_hit = any_hit | hit_u[u]

                @pl.when(any_hit)
                def _process():
                    def ki_loop(ki, _b):
                        pltpu.sync_copy(
                            in_ref.at[c, ki, pl.ds(chi * G, G), :], dbuf)
                        for u in range(G // 16):
                            @pl.when(hit_u[u])
                            def _block(u=u):
                                for t in range(16):
                                    @pl.when(ldsc[u][t] < rows_r)
                                    def _row(u=u, t=t):
                                        ld = ldsc[u][t]
                                        for kk in range(m // 16):
                                            sl = pl.ds(kk * 16, 16)
                                            cur = table[ki, ld, sl]
                                            table[ki, ld, sl] = (
                                                cur + dbuf[u * 16 + t, sl])
                        return _b

                    lax.fori_loop(0, k, ki_loop, None)
                return _a

            lax.fori_loop(0, nch, chunk, None)

            for ki in range(k):
                pltpu.sync_copy(
                    table.at[ki, pl.ds(0, rows_r), :],
                    out_ref.at[c, ki, pl.ds(lo, rows_r), :])
            return _z

        lax.fori_loop(0, np_, one_pass, None)

    return pl.kernel(
        body,
        out_type=jax.ShapeDtypeStruct((2, k, n_out, m), jnp.float32),
        mesh=_sc_mesh(),
        scratch_types=[
            pltpu.VMEM((k, rows_r + 1, m), jnp.float32),
            pltpu.VMEM((G,), jnp.int32),
            pltpu.VMEM((G, m), jnp.float32),
        ],
    )


def _transpose_body(x_ref, w_ref, o_ref):
    o_ref[0, 0] = lax.dot_general(x_ref[...], w_ref[...],
                                  (((0,), (0,)), ((), ())),
                                  preferred_element_type=jnp.float32)


def _transpose_x(x, w0t):
    n = x.shape[1]
    bn = 8192
    return pl.pallas_call(
        _transpose_body,
        grid=(2, n // bn),
        in_specs=[pl.BlockSpec((HALF, bn), lambda h, j: (h, j)),
                  pl.BlockSpec((HALF, 8 * HALF), lambda h, j: (0, 0))],
        out_specs=pl.BlockSpec((1, 1, bn, 8 * HALF), lambda h, j: (h, 0, j, 0)),
        out_shape=jax.ShapeDtypeStruct((2, 1, n, 8 * HALF), jnp.float32),
    )(x, w0t)


def _mix_body(a_ref, w_ref, b_ref, o_ref, *, k_in, k_out):
    acc = None
    for ki in range(k_in):
        part = lax.dot_general(a_ref[0, ki], w_ref[ki],
                               (((1,), (0,)), ((), ())),
                               preferred_element_type=jnp.float32)
        acc = part if acc is None else acc + part
    y = acc + b_ref[0][None, :]
    y = jnp.where(y >= 0, y, 0.2 * y)
    for ko in range(k_out):
        o_ref[0, ko] = y[:, ko * 128:(ko + 1) * 128]


def _mix(agg, w_t, b_big, bn):
    _, k_in, n, m_in = agg.shape
    w_out = w_t.shape[2]
    k_out = w_out // 128
    body = functools.partial(_mix_body, k_in=k_in, k_out=k_out)
    return pl.pallas_call(
        body,
        grid=(2, n // bn),
        in_specs=[
            pl.BlockSpec((1, k_in, bn, m_in), lambda h, j: (h, 0, j, 0)),
            pl.BlockSpec((k_in, m_in, w_out), lambda h, j: (0, 0, 0)),
            pl.BlockSpec((1, w_out), lambda h, j: (0, 0)),
        ],
        out_specs=pl.BlockSpec((1, k_out, bn, 128), lambda h, j: (h, 0, j, 0)),
        out_shape=jax.ShapeDtypeStruct((2, k_out, n, 128), jnp.float32),
    )(agg, w_t, b_big)


def _final_body(a_ref, w_ref, b_ref, m_ref, g_ref, brf_ref, o_ref, *, k_in):
    rows = []
    for h in range(2):
        acc = None
        for ki in range(k_in):
            part = lax.dot_general(a_ref[h, ki], w_ref[ki],
                                   (((1,), (0,)), ((), ())),
                                   preferred_element_type=jnp.float32)
            acc = part if acc is None else acc + part
        y = acc + b_ref[0][None, :]
        y = jnp.where(y >= 0, y, 0.2 * y)
        p = y * m_ref[...]
        rows.append(jnp.sum(p, axis=0, keepdims=True))
    acc = jnp.concatenate(rows, axis=0)
    rf = lax.dot_general(acc, g_ref[...], (((1,), (0,)), ((), ())),
                         preferred_element_type=jnp.float32)
    o_ref[...] = 1.0 / (1.0 + jnp.exp(-(rf + brf_ref[0, 0])))


def _final(agg4, w_t, b_big, m_full, g_sel, b_rf):
    k_in = agg4.shape[1]
    body = functools.partial(_final_body, k_in=k_in)
    return pl.pallas_call(
        body,
        grid=(1,),
        in_specs=[
            pl.BlockSpec(agg4.shape, lambda i: (0, 0, 0, 0)),
            pl.BlockSpec(w_t.shape, lambda i: (0, 0, 0)),
            pl.BlockSpec((1, w_t.shape[2]), lambda i: (0, 0)),
            pl.BlockSpec(m_full.shape, lambda i: (0, 0)),
            pl.BlockSpec(g_sel.shape, lambda i: (0, 0)),
            pl.BlockSpec((1, 1), lambda i: (0, 0)),
        ],
        out_specs=pl.BlockSpec((2, HALF), lambda i: (0, 0)),
        out_shape=jax.ShapeDtypeStruct((2, HALF), jnp.float32),
    )(agg4, w_t, b_big, m_full, g_sel, b_rf)


def kernel(x, dst0, dst1, dst2, dst3, dst4,
           W0, b0, W1, b1, W2, b2, W3, b3, W4, b4, W_rf, b_rf):
    dsts = [dst0, dst1, dst2, dst3, dst4]
    Ws = [W0, W1, W2, W3, W4]
    bs = [b0, b1, b2, b3, b4]

    eye = jnp.eye(HALF, dtype=jnp.float32)
    w0t = jnp.transpose(jnp.kron(eye, W0))
    cur = _transpose_x(x, w0t)
    bns = [4096, 2048, 1024, 512, 128]
    for l in range(5):
        n_out = NODE_SIZES[l + 1]
        _, k, _, m = cur.shape
        agg = _sc_seg(NODE_SIZES[l], n_out, k, m)(
            cur, dsts[l].reshape(-1, 128))
        w_out = HALF * CHANNEL_SIZES[l + 1]
        if l == 0:
            w_t = jnp.eye(w_out, dtype=jnp.float32)[None, :, :]
        else:
            w_in = HALF * CHANNEL_SIZES[l]
            m_in = min(128, w_in)
            k_in = w_in // m_in
            w_big = jnp.kron(eye, Ws[l])
            w_t = jnp.transpose(w_big).reshape(k_in, m_in, w_out)
        b_big = jnp.tile(bs[l], HALF)[None, :]
        if l < 4:
            cur = _mix(agg, w_t, b_big, bns[l])
        else:
            n4, c4 = NODE_SIZES[5], CHANNEL_SIZES[5]
            wrf = W_rf[:, 0].reshape(c4, n4)
            m_full = jnp.tile(jnp.transpose(wrf), (1, HALF))
            g_sel = jnp.repeat(jnp.eye(HALF, dtype=jnp.float32),
                               c4, axis=0)
            rf = _final(agg, w_t, b_big, m_full, g_sel, b_rf.reshape(1, 1))
    rf = rf.reshape(32, 1)
    neg = jnp.full((32,), -1.0, dtype=jnp.float32)
    return (rf, neg, neg, neg)


# speedup vs baseline: 1.2448x; 1.0022x over previous
import functools

import jax
import jax.numpy as jnp
from jax import lax
from jax.experimental import pallas as pl
from jax.experimental.pallas import tpu as pltpu
from jax.experimental.pallas import tpu_sc as plsc

NODE_SIZES = [131072, 32768, 8192, 2048, 512, 128]
CHANNEL_SIZES = [1, 8, 16, 32, 64, 128]
NS = 16
HALF = 16


CH = 512
G = 128


def _sc_mesh():
    return plsc.VectorSubcoreMesh(core_axis_name="c", subcore_axis_name="s",
                                  num_cores=2, num_subcores=NS)


@functools.cache
def _sc_seg(n_in: int, n_out: int, k: int, m: int):
    w = k * m
    np_ = 1
    while (n_out // (NS * np_) + 1) * w > 100000:
        np_ *= 2
    rows_r = n_out // (NS * np_)
    nch = n_in // G
    IW = min(16, nch)

    def body(in_ref, dst_ref, out_ref, table, idxw, dbuf, sem):
        c = lax.axis_index("c")
        s = lax.axis_index("s")
        zero16 = jnp.zeros((16,), jnp.float32)

        def one_pass(p, _z):
            lo = (s * np_ + p) * rows_r

            def zf(i, _):
                q = i // (m // 16)
                zk = i % (m // 16)
                kz = q // (rows_r + 1)
                zr = q % (rows_r + 1)
                table[kz, zr, pl.ds(zk * 16, 16)] = zero16
                return _

            lax.fori_loop(0, k * (rows_r + 1) * (m // 16), zf, None)

            def chunk(chi, _a):
                ldsc, hit_u = [], []
                for u in range(G // 16):
                    v = idxw[lax.rem(chi, IW), pl.ds(u * 16, 16)]
                    mu = (v >= lo) & (v < lo + rows_r)
                    ldv = jnp.where(mu, v - lo, rows_r)
                    row = [ldv[t] for t in range(16)]
                    ldsc.append(row)
                    hu = row[0] < rows_r
                    for t in range(1, 16):
                        hu = hu | (row[t] < rows_r)
                    hit_u.append(hu)
                any_hit = hit_u[0]
                for u in range(1, G // 16):
                    any_hit = any_hit | hit_u[u]

                @pl.when(any_hit)
                def _process():
                    def ki_loop(ki, _b):
                        pltpu.sync_copy(
                            in_ref.at[c, ki, pl.ds(chi * G, G), :], dbuf)
                        for u in range(G // 16):
                            @pl.when(hit_u[u])
                            def _block(u=u):
                                for t in range(16):
                                    @pl.when(ldsc[u][t] < rows_r)
                                    def _row(u=u, t=t):
                                        ld = ldsc[u][t]
                                        for kk in range(m // 16):
                                            sl = pl.ds(kk * 16, 16)
                                            cur = table[ki, ld, sl]
                                            table[ki, ld, sl] = (
                                                cur + dbuf[u * 16 + t, sl])
                        return _b

                    lax.fori_loop(0, k, ki_loop, None)
                return _a

            def ichunk(wi, _a):
                pltpu.async_copy(dst_ref.at[pl.ds(wi * IW, IW), :], idxw,
                                 sem).wait()
                lax.fori_loop(wi * IW, (wi + 1) * IW, chunk, None)
                return _a

            lax.fori_loop(0, nch // IW, ichunk, None)

            for ki in range(k):
                pltpu.sync_copy(
                    table.at[ki, pl.ds(0, rows_r), :],
                    out_ref.at[c, ki, pl.ds(lo, rows_r), :])
            return _z

        lax.fori_loop(0, np_, one_pass, None)

    return pl.kernel(
        body,
        out_type=jax.ShapeDtypeStruct((2, k, n_out, m), jnp.float32),
        mesh=_sc_mesh(),
        scratch_types=[
            pltpu.VMEM((k, rows_r + 1, m), jnp.float32),
            pltpu.VMEM((IW, G), jnp.int32),
            pltpu.VMEM((G, m), jnp.float32),
            pltpu.SemaphoreType.DMA,
        ],
    )


def _transpose_body(x_ref, w_ref, o_ref):
    o_ref[0, 0] = lax.dot_general(x_ref[...], w_ref[...],
                                  (((0,), (0,)), ((), ())),
                                  preferred_element_type=jnp.float32)


def _transpose_x(x, w0t):
    n = x.shape[1]
    bn = 8192
    return pl.pallas_call(
        _transpose_body,
        grid=(2, n // bn),
        in_specs=[pl.BlockSpec((HALF, bn), lambda h, j: (h, j)),
                  pl.BlockSpec((HALF, 8 * HALF), lambda h, j: (0, 0))],
        out_specs=pl.BlockSpec((1, 1, bn, 8 * HALF), lambda h, j: (h, 0, j, 0)),
        out_shape=jax.ShapeDtypeStruct((2, 1, n, 8 * HALF), jnp.float32),
    )(x, w0t)


def _mix_body(a_ref, w_ref, b_ref, o_ref, *, k_in, k_out):
    acc = None
    for ki in range(k_in):
        part = lax.dot_general(a_ref[0, ki], w_ref[ki],
                               (((1,), (0,)), ((), ())),
                               preferred_element_type=jnp.float32)
        acc = part if acc is None else acc + part
    y = acc + b_ref[0][None, :]
    y = jnp.where(y >= 0, y, 0.2 * y)
    for ko in range(k_out):
        o_ref[0, ko] = y[:, ko * 128:(ko + 1) * 128]


def _mix(agg, w_t, b_big, bn):
    _, k_in, n, m_in = agg.shape
    w_out = w_t.shape[2]
    k_out = w_out // 128
    body = functools.partial(_mix_body, k_in=k_in, k_out=k_out)
    return pl.pallas_call(
        body,
        grid=(2, n // bn),
        in_specs=[
            pl.BlockSpec((1, k_in, bn, m_in), lambda h, j: (h, 0, j, 0)),
            pl.BlockSpec((k_in, m_in, w_out), lambda h, j: (0, 0, 0)),
            pl.BlockSpec((1, w_out), lambda h, j: (0, 0)),
        ],
        out_specs=pl.BlockSpec((1, k_out, bn, 128), lambda h, j: (h, 0, j, 0)),
        out_shape=jax.ShapeDtypeStruct((2, k_out, n, 128), jnp.float32),
    )(agg, w_t, b_big)


def _final_body(a_ref, w_ref, b_ref, m_ref, g_ref, brf_ref, o_ref, *, k_in):
    rows = []
    for h in range(2):
        acc = None
        for ki in range(k_in):
            part = lax.dot_general(a_ref[h, ki], w_ref[ki],
                                   (((1,), (0,)), ((), ())),
                                   preferred_element_type=jnp.float32)
            acc = part if acc is None else acc + part
        y = acc + b_ref[0][None, :]
        y = jnp.where(y >= 0, y, 0.2 * y)
        p = y * m_ref[...]
        rows.append(jnp.sum(p, axis=0, keepdims=True))
    acc = jnp.concatenate(rows, axis=0)
    rf = lax.dot_general(acc, g_ref[...], (((1,), (0,)), ((), ())),
                         preferred_element_type=jnp.float32)
    o_ref[...] = 1.0 / (1.0 + jnp.exp(-(rf + brf_ref[0, 0])))


def _final(agg4, w_t, b_big, m_full, g_sel, b_rf):
    k_in = agg4.shape[1]
    body = functools.partial(_final_body, k_in=k_in)
    return pl.pallas_call(
        body,
        grid=(1,),
        in_specs=[
            pl.BlockSpec(agg4.shape, lambda i: (0, 0, 0, 0)),
            pl.BlockSpec(w_t.shape, lambda i: (0, 0, 0)),
            pl.BlockSpec((1, w_t.shape[2]), lambda i: (0, 0)),
            pl.BlockSpec(m_full.shape, lambda i: (0, 0)),
            pl.BlockSpec(g_sel.shape, lambda i: (0, 0)),
            pl.BlockSpec((1, 1), lambda i: (0, 0)),
        ],
        out_specs=pl.BlockSpec((2, HALF), lambda i: (0, 0)),
        out_shape=jax.ShapeDtypeStruct((2, HALF), jnp.float32),
    )(agg4, w_t, b_big, m_full, g_sel, b_rf)


def kernel(x, dst0, dst1, dst2, dst3, dst4,
           W0, b0, W1, b1, W2, b2, W3, b3, W4, b4, W_rf, b_rf):
    dsts = [dst0, dst1, dst2, dst3, dst4]
    Ws = [W0, W1, W2, W3, W4]
    bs = [b0, b1, b2, b3, b4]

    eye = jnp.eye(HALF, dtype=jnp.float32)
    w0t = jnp.transpose(jnp.kron(eye, W0))
    cur = _transpose_x(x, w0t)
    bns = [4096, 2048, 1024, 512, 128]
    for l in range(5):
        n_out = NODE_SIZES[l + 1]
        _, k, _, m = cur.shape
        agg = _sc_seg(NODE_SIZES[l], n_out, k, m)(
            cur, dsts[l].reshape(-1, 128))
        w_out = HALF * CHANNEL_SIZES[l + 1]
        if l == 0:
            w_t = jnp.eye(w_out, dtype=jnp.float32)[None, :, :]
        else:
            w_in = HALF * CHANNEL_SIZES[l]
            m_in = min(128, w_in)
            k_in = w_in // m_in
            w_big = jnp.kron(eye, Ws[l])
            w_t = jnp.transpose(w_big).reshape(k_in, m_in, w_out)
        b_big = jnp.tile(bs[l], HALF)[None, :]
        if l < 4:
            cur = _mix(agg, w_t, b_big, bns[l])
        else:
            n4, c4 = NODE_SIZES[5], CHANNEL_SIZES[5]
            wrf = W_rf[:, 0].reshape(c4, n4)
            m_full = jnp.tile(jnp.transpose(wrf), (1, HALF))
            g_sel = jnp.repeat(jnp.eye(HALF, dtype=jnp.float32),
                               c4, axis=0)
            rf = _final(agg, w_t, b_big, m_full, g_sel, b_rf.reshape(1, 1))
    rf = rf.reshape(32, 1)
    neg = jnp.full((32,), -1.0, dtype=jnp.float32)
    return (rf, neg, neg, neg)


# speedup vs baseline: 1.2469x; 1.0017x over previous
import functools

import jax
import jax.numpy as jnp
from jax import lax
from jax.experimental import pallas as pl
from jax.experimental.pallas import tpu as pltpu
from jax.experimental.pallas import tpu_sc as plsc

NODE_SIZES = [131072, 32768, 8192, 2048, 512, 128]
CHANNEL_SIZES = [1, 8, 16, 32, 64, 128]
NS = 16
HALF = 16


CH = 512
G = 128


def _sc_mesh():
    return plsc.VectorSubcoreMesh(core_axis_name="c", subcore_axis_name="s",
                                  num_cores=2, num_subcores=NS)


@functools.cache
def _sc_seg(n_in: int, n_out: int, k: int, m: int):
    w = k * m
    np_ = 1
    while (n_out // (NS * np_) + 1) * w > 100000:
        np_ *= 2
    rows_r = n_out // (NS * np_)
    nch = n_in // G
    IW = min(16, nch)

    def body(in_ref, dst_ref, out_ref, table, idxw, dbuf, sem):
        c = lax.axis_index("c")
        s = lax.axis_index("s")
        zero16 = jnp.zeros((16,), jnp.float32)

        def one_pass(p, _z):
            lo = (s * np_ + p) * rows_r

            def zf(i, _):
                q = i // (m // 16)
                zk = i % (m // 16)
                kz = q // (rows_r + 1)
                zr = q % (rows_r + 1)
                table[kz, zr, pl.ds(zk * 16, 16)] = zero16
                return _

            lax.fori_loop(0, k * (rows_r + 1) * (m // 16), zf, None)

            def chunk(chi, _a):
                ldsc, hit_u = [], []
                for u in range(G // 16):
                    v = idxw[lax.rem(chi, IW), pl.ds(u * 16, 16)]
                    mu = (v >= lo) & (v < lo + rows_r)
                    ldv = jnp.where(mu, v - lo, rows_r)
                    row = [ldv[t] for t in range(16)]
                    ldsc.append(row)
                    hu = row[0] < rows_r
                    for t in range(1, 16):
                        hu = hu | (row[t] < rows_r)
                    hit_u.append(hu)
                any_hit = hit_u[0]
                for u in range(1, G // 16):
                    any_hit = any_hit | hit_u[u]

                @pl.when(any_hit)
                def _process():
                    def ki_loop(ki, _b):
                        pltpu.sync_copy(
                            in_ref.at[c, ki, pl.ds(chi * G, G), :], dbuf)
                        for u in range(G // 16):
                            @pl.when(hit_u[u])
                            def _block(u=u):
                                for t in range(16):
                                    @pl.when(ldsc[u][t] < rows_r)
                                    def _row(u=u, t=t):
                                        ld = ldsc[u][t]
                                        for kk in range(m // 16):
                                            sl = pl.ds(kk * 16, 16)
                                            cur = table[ki, ld, sl]
                                            table[ki, ld, sl] = (
                                                cur + dbuf[u * 16 + t, sl])
                        return _b

                    lax.fori_loop(0, k, ki_loop, None)
                return _a

            def ichunk(wi, _a):
                we = lax.rem(wi + s * max(1, (nch // IW) // NS), nch // IW)
                pltpu.async_copy(dst_ref.at[pl.ds(we * IW, IW), :], idxw,
                                 sem).wait()
                lax.fori_loop(we * IW, (we + 1) * IW, chunk, None)
                return _a

            lax.fori_loop(0, nch // IW, ichunk, None)

            for ki in range(k):
                pltpu.sync_copy(
                    table.at[ki, pl.ds(0, rows_r), :],
                    out_ref.at[c, ki, pl.ds(lo, rows_r), :])
            return _z

        lax.fori_loop(0, np_, one_pass, None)

    return pl.kernel(
        body,
        out_type=jax.ShapeDtypeStruct((2, k, n_out, m), jnp.float32),
        mesh=_sc_mesh(),
        scratch_types=[
            pltpu.VMEM((k, rows_r + 1, m), jnp.float32),
            pltpu.VMEM((IW, G), jnp.int32),
            pltpu.VMEM((G, m), jnp.float32),
            pltpu.SemaphoreType.DMA,
        ],
    )


def _transpose_body(x_ref, w_ref, o_ref):
    o_ref[0, 0] = lax.dot_general(x_ref[...], w_ref[...],
                                  (((0,), (0,)), ((), ())),
                                  preferred_element_type=jnp.float32)


def _transpose_x(x, w0t):
    n = x.shape[1]
    bn = 8192
    return pl.pallas_call(
        _transpose_body,
        grid=(2, n // bn),
        in_specs=[pl.BlockSpec((HALF, bn), lambda h, j: (h, j)),
                  pl.BlockSpec((HALF, 8 * HALF), lambda h, j: (0, 0))],
        out_specs=pl.BlockSpec((1, 1, bn, 8 * HALF), lambda h, j: (h, 0, j, 0)),
        out_shape=jax.ShapeDtypeStruct((2, 1, n, 8 * HALF), jnp.float32),
    )(x, w0t)


def _mix_body(a_ref, w_ref, b_ref, o_ref, *, k_in, k_out):
    acc = None
    for ki in range(k_in):
        part = lax.dot_general(a_ref[0, ki], w_ref[ki],
                               (((1,), (0,)), ((), ())),
                               preferred_element_type=jnp.float32)
        acc = part if acc is None else acc + part
    y = acc + b_ref[0][None, :]
    y = jnp.where(y >= 0, y, 0.2 * y)
    for ko in range(k_out):
        o_ref[0, ko] = y[:, ko * 128:(ko + 1) * 128]


def _mix(agg, w_t, b_big, bn):
    _, k_in, n, m_in = agg.shape
    w_out = w_t.shape[2]
    k_out = w_out // 128
    body = functools.partial(_mix_body, k_in=k_in, k_out=k_out)
    return pl.pallas_call(
        body,
        grid=(2, n // bn),
        in_specs=[
            pl.BlockSpec((1, k_in, bn, m_in), lambda h, j: (h, 0, j, 0)),
            pl.BlockSpec((k_in, m_in, w_out), lambda h, j: (0, 0, 0)),
            pl.BlockSpec((1, w_out), lambda h, j: (0, 0)),
        ],
        out_specs=pl.BlockSpec((1, k_out, bn, 128), lambda h, j: (h, 0, j, 0)),
        out_shape=jax.ShapeDtypeStruct((2, k_out, n, 128), jnp.float32),
    )(agg, w_t, b_big)


def _final_body(a_ref, w_ref, b_ref, m_ref, g_ref, brf_ref, o_ref, *, k_in):
    rows = []
    for h in range(2):
        acc = None
        for ki in range(k_in):
            part = lax.dot_general(a_ref[h, ki], w_ref[ki],
                                   (((1,), (0,)), ((), ())),
                                   preferred_element_type=jnp.float32)
            acc = part if acc is None else acc + part
        y = acc + b_ref[0][None, :]
        y = jnp.where(y >= 0, y, 0.2 * y)
        p = y * m_ref[...]
        rows.append(jnp.sum(p, axis=0, keepdims=True))
    acc = jnp.concatenate(rows, axis=0)
    rf = lax.dot_general(acc, g_ref[...], (((1,), (0,)), ((), ())),
                         preferred_element_type=jnp.float32)
    o_ref[...] = 1.0 / (1.0 + jnp.exp(-(rf + brf_ref[0, 0])))


def _final(agg4, w_t, b_big, m_full, g_sel, b_rf):
    k_in = agg4.shape[1]
    body = functools.partial(_final_body, k_in=k_in)
    return pl.pallas_call(
        body,
        grid=(1,),
        in_specs=[
            pl.BlockSpec(agg4.shape, lambda i: (0, 0, 0, 0)),
            pl.BlockSpec(w_t.shape, lambda i: (0, 0, 0)),
            pl.BlockSpec((1, w_t.shape[2]), lambda i: (0, 0)),
            pl.BlockSpec(m_full.shape, lambda i: (0, 0)),
            pl.BlockSpec(g_sel.shape, lambda i: (0, 0)),
            pl.BlockSpec((1, 1), lambda i: (0, 0)),
        ],
        out_specs=pl.BlockSpec((2, HALF), lambda i: (0, 0)),
        out_shape=jax.ShapeDtypeStruct((2, HALF), jnp.float32),
    )(agg4, w_t, b_big, m_full, g_sel, b_rf)


def kernel(x, dst0, dst1, dst2, dst3, dst4,
           W0, b0, W1, b1, W2, b2, W3, b3, W4, b4, W_rf, b_rf):
    dsts = [dst0, dst1, dst2, dst3, dst4]
    Ws = [W0, W1, W2, W3, W4]
    bs = [b0, b1, b2, b3, b4]

    eye = jnp.eye(HALF, dtype=jnp.float32)
    w0t = jnp.transpose(jnp.kron(eye, W0))
    cur = _transpose_x(x, w0t)
    bns = [4096, 2048, 1024, 512, 128]
    for l in range(5):
        n_out = NODE_SIZES[l + 1]
        _, k, _, m = cur.shape
        agg = _sc_seg(NODE_SIZES[l], n_out, k, m)(
            cur, dsts[l].reshape(-1, 128))
        w_out = HALF * CHANNEL_SIZES[l + 1]
        if l == 0:
            w_t = jnp.eye(w_out, dtype=jnp.float32)[None, :, :]
        else:
            w_in = HALF * CHANNEL_SIZES[l]
            m_in = min(128, w_in)
            k_in = w_in // m_in
            w_big = jnp.kron(eye, Ws[l])
            w_t = jnp.transpose(w_big).reshape(k_in, m_in, w_out)
        b_big = jnp.tile(bs[l], HALF)[None, :]
        if l < 4:
            cur = _mix(agg, w_t, b_big, bns[l])
        else:
            n4, c4 = NODE_SIZES[5], CHANNEL_SIZES[5]
            wrf = W_rf[:, 0].reshape(c4, n4)
            m_full = jnp.tile(jnp.transpose(wrf), (1, HALF))
            g_sel = jnp.repeat(jnp.eye(HALF, dtype=jnp.float32),
                               c4, axis=0)
            rf = _final(agg, w_t, b_big, m_full, g_sel, b_rf.reshape(1, 1))
    rf = rf.reshape(32, 1)
    neg = jnp.full((32,), -1.0, dtype=jnp.float32)
    return (rf, neg, neg, neg)
